# baseline (speedup 1.0000x reference)
import functools

import jax
import jax.numpy as jnp
from jax import lax
from jax.experimental import pallas as pl
from jax.experimental.pallas import tpu as pltpu
from jax.experimental.pallas import tpu_sc as plsc

N = 10000
E = 320000
D = 128
NC = 2
NS = 16
L = 16
NW = NC * NS
EPW = E // NW
B = 80
K = EPW // B
N_PAD = 10240
RPT = N_PAD // NS

NROW = 4
GLEAD = 3
NIDX = 8
ILEAD = 6


def _matmul_body(x_ref, w_ref, b_ref, o_ref):
    o_ref[...] = (
        jnp.dot(x_ref[...], w_ref[...], preferred_element_type=jnp.float32)
        + b_ref[...]
    )


def _value_proj(x, w, b):
    return pl.pallas_call(
        _matmul_body,
        out_shape=jax.ShapeDtypeStruct((N, D), jnp.float32),
    )(x, w, b)


def _sc_body(v_hbm, da_hbm, out_hbm,
             rows0, rows1, rows2, rows3,
             da0, da1, da2, da3, da4, da5, da6, da7,
             acc_sh,
             semg0, semg1, semg2, semg3,
             sems0, sems1, sems2, sems3,
             semi0, semi1, semi2, semi3, semi4, semi5, semi6, semi7):
    rows = [rows0, rows1, rows2, rows3]
    dab = [da0, da1, da2, da3, da4, da5, da6, da7]
    semg = [semg0, semg1, semg2, semg3]
    sems = [sems0, sems1, sems2, sems3]
    semi = [semi0, semi1, semi2, semi3, semi4, semi5, semi6, semi7]
    cid = lax.axis_index("c")
    sid = lax.axis_index("s")
    wid = sid * NC + cid
    tile_base = sid * RPT

    def _fire_idx(s, ci):
        pltpu.async_copy(da_hbm.at[wid, ci], dab[s], semi[s])

    def _wait_idx(s, ci):
        pltpu.make_async_copy(da_hbm.at[wid, ci], dab[s], semi[s]).wait()

    def _fire_gather(rb, s, ci):
        pltpu.async_copy(v_hbm.at[dab[s].at[0]], rows[rb], semg[rb])

    def _wait_gather(rb, s, ci):
        pltpu.make_async_copy(v_hbm.at[dab[s].at[0]], rows[rb], semg[rb]).wait()

    def _fire_scatter(rb, s):
        pltpu.async_copy(rows[rb], acc_sh.at[dab[s].at[2]], sems[rb], add=True)

    def _wait_scatter(rb, s):
        pltpu.make_async_copy(rows[rb], acc_sh.at[dab[s].at[2]], sems[rb]).wait()

    def _zero_row(i, carry):
        for j in range(D // L):
            rows0[i, pl.ds(j * L, L)] = jnp.zeros((L,), jnp.float32)
        return carry

    lax.fori_loop(0, B, _zero_row, 0)
    for k in range(RPT // B):
        pltpu.sync_copy(rows0, acc_sh.at[pl.ds(tile_base + k * B, B)])

    for s in range(ILEAD):
        _fire_idx(s, s)
    for rb in range(GLEAD):
        _wait_idx(rb, rb)
        _fire_gather(rb, rb, rb)

    plsc.subcore_barrier()

    def _scale(rbuf, dslot):
        def _scale_group(g, c2):
            av16 = lax.bitcast_convert_type(dslot[1, pl.ds(g * L, L)], jnp.float32)
            for lane in range(L):
                a = lax.gather(
                    av16, jnp.full((L, 1), lane, jnp.int32),
                    lax.GatherDimensionNumbers(
                        offset_dims=(), collapsed_slice_dims=(0,),
                        start_index_map=(0,)),
                    slice_sizes=(1,),
                    mode=lax.GatherScatterMode.PROMISE_IN_BOUNDS)
                e = g * L + lane
                for j in range(D // L):
                    rbuf[e, pl.ds(j * L, L)] = rbuf[e, pl.ds(j * L, L)] * a
            return c2

        lax.fori_loop(0, B // L, _scale_group, 0)

    def _outer(g, carry):
        for cc in range(NIDX):
            ci = g * NIDX + cc
            rb = cc % NROW
            rb3 = (cc + GLEAD) % NROW
            s3 = (cc + GLEAD) % NIDX
            s6 = (cc + ILEAD) % NIDX

            @pl.when(ci < K)
            def _consume():
                _wait_gather(rb, cc, ci)
                _scale(rows[rb], dab[cc])
                _fire_scatter(rb, cc)

            @pl.when(ci + ILEAD < K)
            def _refill():
                _fire_idx(s6, ci + ILEAD)

            @pl.when((ci >= 1) & (ci + GLEAD < K))
            def _drain_prev():
                _wait_scatter(rb3, (cc + NIDX - 1) % NIDX)

            @pl.when(ci + GLEAD < K)
            def _gather_ahead():
                _wait_idx(s3, ci + GLEAD)
                _fire_gather(rb3, s3, ci + GLEAD)
        return carry

    lax.fori_loop(0, K // NIDX + 1, _outer, 0)

    for c in range(K - NROW, K):
        _wait_scatter(c % NROW, c % NIDX)

    plsc.subcore_barrier()

    pltpu.sync_copy(
        acc_sh.at[pl.ds(tile_base, RPT)],
        out_hbm.at[cid, pl.ds(tile_base, RPT)],
    )


_sc_agg = functools.partial(
    pl.kernel,
    out_type=jax.ShapeDtypeStruct((NC, N_PAD, D), jnp.float32),
    mesh=plsc.VectorSubcoreMesh(
        core_axis_name="c", subcore_axis_name="s",
        num_cores=NC, num_subcores=NS,
    ),
    scratch_types=(
        [pltpu.VMEM((B, D), jnp.float32)] * NROW
        + [pltpu.VMEM((3, B), jnp.int32)] * NIDX
        + [pltpu.VMEM_SHARED((N_PAD, D), jnp.float32)]
        + [pltpu.SemaphoreType.DMA] * (2 * NROW + NIDX)
    ),
)(_sc_body)


def _bn_body(p_ref, g_ref, b_ref, o_ref):
    s = p_ref[0, :N, :] + p_ref[1, :N, :]
    mean = jnp.mean(s, axis=0, keepdims=True)
    var = jnp.mean(jnp.square(s - mean), axis=0, keepdims=True)
    o = (s - mean) * jax.lax.rsqrt(var + 1e-5) * g_ref[...] + b_ref[...]
    o_ref[...] = jnp.where(o >= 0, o, 0.01 * o)


def _bn_leaky(parts, gamma, beta):
    return pl.pallas_call(
        _bn_body,
        out_shape=jax.ShapeDtypeStruct((N, D), jnp.float32),
    )(parts, gamma, beta)


def kernel(x, src, dst, att_score, Wv, bv, gamma, beta):
    w_cat = jnp.transpose(Wv, (1, 0, 2)).reshape(D, D)
    b_cat = bv.reshape(1, D)
    v = _value_proj(x, w_cat, b_cat)
    att_bits = lax.bitcast_convert_type(att_score.reshape(E), jnp.int32)
    da = jnp.stack(
        [dst.reshape(NW, K, B), att_bits.reshape(NW, K, B),
         src.reshape(NW, K, B)], axis=2)
    parts = _sc_agg(v, da)
    return _bn_leaky(parts, gamma.reshape(1, D), beta.reshape(1, D))
